# baseline (speedup 1.0000x reference)
import functools

import jax
import jax.numpy as jnp
from jax import lax
from jax.experimental import pallas as pl
from jax.experimental.pallas import tpu as pltpu
from jax.experimental.pallas import tpu_sc as plsc

_N_CTX = 4
_SEQ = 77
_DIM = 512
_KEEP = _SEQ - _N_CTX
_PAD = 80


def _sc_prompt_gather(idx_pad, table, ctx):
    n_cls = idx_pad.shape[0]
    info = plsc.get_sparse_core_info()
    nw = info.num_cores * info.num_subcores
    mesh = plsc.VectorSubcoreMesh(core_axis_name="c", subcore_axis_name="s")

    @functools.partial(
        pl.kernel,
        mesh=mesh,
        compiler_params=pltpu.CompilerParams(use_tc_tiling_on_sc=False),
        out_type=jax.ShapeDtypeStruct((n_cls, _SEQ, _DIM), jnp.float32),
        scratch_types=[
            pltpu.VMEM((_PAD,), jnp.int32),
            pltpu.VMEM((_PAD, _DIM), jnp.float32),
            pltpu.VMEM((_N_CTX, _DIM), jnp.float32),
            pltpu.SemaphoreType.DMA,
        ],
    )
    def k(idx_hbm, table_hbm, ctx_hbm, out_hbm, idx_v, rows_v, ctx_v, sem):
        wid = lax.axis_index("s") * info.num_cores + lax.axis_index("c")
        pltpu.sync_copy(ctx_hbm, ctx_v)
        nfull = n_cls // nw
        rem = n_cls % nw
        nrounds = nfull + (wid < rem).astype(jnp.int32)

        def body(r, _):
            c = r * nw + wid
            pltpu.sync_copy(idx_hbm.at[c], idx_v)
            pltpu.async_copy(table_hbm.at[idx_v], rows_v, sem).wait()
            pltpu.sync_copy(rows_v.at[pl.ds(0, 1)], out_hbm.at[c, pl.ds(0, 1)])
            pltpu.sync_copy(ctx_v, out_hbm.at[c, pl.ds(1, _N_CTX)])
            pltpu.sync_copy(
                rows_v.at[pl.ds(1, _KEEP - 1)],
                out_hbm.at[c, pl.ds(1 + _N_CTX, _KEEP - 1)],
            )
            return _

        lax.fori_loop(0, nrounds, body, None)

    return k(idx_pad, table, ctx)


def kernel(tokenized_prompts, token_embedding, ctx):
    n_cls = tokenized_prompts.shape[0]
    idx_pad = jnp.concatenate(
        [
            tokenized_prompts[:, :1],
            tokenized_prompts[:, 1 + _N_CTX:],
            jnp.zeros((n_cls, _PAD - _KEEP), jnp.int32),
        ],
        axis=1,
    )
    return _sc_prompt_gather(idx_pad, token_embedding, ctx)


# speedup vs baseline: 1.5043x; 1.5043x over previous
import functools

import jax
import jax.numpy as jnp
from jax import lax
from jax.experimental import pallas as pl
from jax.experimental.pallas import tpu as pltpu
from jax.experimental.pallas import tpu_sc as plsc

_N_CTX = 4
_SEQ = 77
_DIM = 512
_SUF = _SEQ - 1 - _N_CTX
_IDXROW = 80


def _sc_prompt_gather(idx_block, table, ctx, n_cls):
    info = plsc.get_sparse_core_info()
    nw = info.num_cores * info.num_subcores
    rpw = idx_block.shape[1] - 1
    nfull = n_cls // nw
    rem = n_cls % nw
    mesh = plsc.VectorSubcoreMesh(core_axis_name="c", subcore_axis_name="s")

    @functools.partial(
        pl.kernel,
        mesh=mesh,
        compiler_params=pltpu.CompilerParams(use_tc_tiling_on_sc=False),
        out_type=jax.ShapeDtypeStruct((n_cls, _SEQ, _DIM), jnp.float32),
        scratch_types=[
            pltpu.VMEM((rpw + 1, _IDXROW), jnp.int32),
            pltpu.VMEM((rpw, _DIM), jnp.float32),
            pltpu.VMEM((_N_CTX, _DIM), jnp.float32),
            pltpu.VMEM((_SUF, _DIM), jnp.float32),
            pltpu.VMEM((_SUF, _DIM), jnp.float32),
            pltpu.SemaphoreType.DMA,
            pltpu.SemaphoreType.DMA,
            pltpu.SemaphoreType.DMA,
        ],
    )
    def k(idx_hbm, table_hbm, ctx_hbm, out_hbm,
          idx_v, pbuf, ctx_v, suf0, suf1, gs0, gs1, ws):
        wid = lax.axis_index("s") * info.num_cores + lax.axis_index("c")
        nr = nfull + (wid < rem).astype(jnp.int32)

        pltpu.sync_copy(idx_hbm.at[wid], idx_v)
        pltpu.sync_copy(ctx_hbm, ctx_v)
        pltpu.async_copy(
            table_hbm.at[idx_v.at[0, pl.ds(0, rpw)]], pbuf, gs0).wait()

        def issue_gather(r, buf, sem):
            pltpu.async_copy(
                table_hbm.at[idx_v.at[1 + r, pl.ds(0, _SUF)]], buf, sem)

        def write_round(r, buf):
            c = r * nw + wid
            w1 = pltpu.make_async_copy(
                pbuf.at[pl.ds(r, 1)], out_hbm.at[c, pl.ds(0, 1)], ws)
            w1.start()
            w2 = pltpu.make_async_copy(
                ctx_v, out_hbm.at[c, pl.ds(1, _N_CTX)], ws)
            w2.start()
            w3 = pltpu.make_async_copy(
                buf, out_hbm.at[c, pl.ds(1 + _N_CTX, _SUF)], ws)
            w3.start()
            w1.wait()
            w2.wait()
            w3.wait()

        def wait_gather(r, buf, sem):
            pltpu.make_async_copy(
                table_hbm.at[idx_v.at[1 + r, pl.ds(0, _SUF)]], buf, sem
            ).wait()

        issue_gather(0, suf0, gs0)

        def body(rr, _):
            r0 = rr * 2
            r1 = r0 + 1
            wait_gather(r0, suf0, gs0)

            @pl.when(r1 < nr)
            def _():
                issue_gather(r1, suf1, gs1)

            write_round(r0, suf0)

            @pl.when(r1 < nr)
            def _():
                wait_gather(r1, suf1, gs1)

                @pl.when(r1 + 1 < nr)
                def _():
                    issue_gather(r1 + 1, suf0, gs0)

                write_round(r1, suf1)

            return _

        lax.fori_loop(0, (rpw + 1) // 2, body, None)

    return k(idx_block, table, ctx)


def kernel(tokenized_prompts, token_embedding, ctx):
    n_cls = tokenized_prompts.shape[0]
    info = plsc.get_sparse_core_info()
    nw = info.num_cores * info.num_subcores
    rpw = -(-n_cls // nw)
    pad = nw * rpw - n_cls

    tokp = jnp.concatenate(
        [tokenized_prompts,
         jnp.zeros((pad, tokenized_prompts.shape[1]), jnp.int32)], axis=0)
    by_worker = tokp.reshape(rpw, nw, _SEQ).transpose(1, 0, 2)
    prefix_row = jnp.concatenate(
        [by_worker[:, :, 0], jnp.zeros((nw, _IDXROW - rpw), jnp.int32)],
        axis=1)[:, None, :]
    suffix_rows = jnp.concatenate(
        [by_worker[:, :, 1 + _N_CTX:],
         jnp.zeros((nw, rpw, _IDXROW - _SUF), jnp.int32)], axis=2)
    idx_block = jnp.concatenate([prefix_row, suffix_rows], axis=1)

    return _sc_prompt_gather(idx_block, token_embedding, ctx, n_cls)


# speedup vs baseline: 5.5192x; 3.6690x over previous
import functools

import jax
import jax.numpy as jnp
from jax import lax
from jax.experimental import pallas as pl
from jax.experimental.pallas import tpu as pltpu
from jax.experimental.pallas import tpu_sc as plsc

_N_CTX = 4
_SEQ = 77
_DIM = 512
_IDXROW = 80
_LANES = 16


def _sc_prompt_gather(idx_flat, table, ctx_flat, n_cls):
    info = plsc.get_sparse_core_info()
    nw = info.num_cores * info.num_subcores
    rpw = idx_flat.shape[0] // (nw * _IDXROW)
    nfull = n_cls // nw
    rem = n_cls % nw
    mesh = plsc.VectorSubcoreMesh(core_axis_name="c", subcore_axis_name="s")

    @functools.partial(
        pl.kernel,
        mesh=mesh,
        out_type=jax.ShapeDtypeStruct((n_cls, _SEQ, _DIM), jnp.float32),
        scratch_types=[
            pltpu.VMEM((rpw * _IDXROW,), jnp.int32),
            pltpu.VMEM((_N_CTX * _DIM,), jnp.float32),
            pltpu.VMEM((_SEQ, _DIM), jnp.float32),
            pltpu.VMEM((_SEQ, _DIM), jnp.float32),
            pltpu.SemaphoreType.DMA,
            pltpu.SemaphoreType.DMA,
        ],
    )
    def k(idx_hbm, table_hbm, ctx_hbm, out_hbm,
          idx_v, ctx_v, row0, row1, gs0, gs1):
        wid = lax.axis_index("s") * info.num_cores + lax.axis_index("c")
        nr = nfull + (wid < rem).astype(jnp.int32)

        pltpu.sync_copy(idx_hbm.at[pl.ds(wid * (rpw * _IDXROW),
                                         rpw * _IDXROW)], idx_v)
        pltpu.sync_copy(ctx_hbm, ctx_v)

        def idx_slice(r):
            return idx_v.at[pl.ds(r * _IDXROW, _SEQ)]

        def issue_gather(r, buf, sem):
            pltpu.async_copy(table_hbm.at[idx_slice(r)], buf, sem)

        def wait_gather(buf, sem):
            pltpu.make_async_copy(table_hbm.at[idx_slice(0)], buf, sem).wait()

        def round_sect(r, buf, sem, obuf, osem):
            wait_gather(buf, sem)

            @pl.when(r + 1 < nr)
            def _():
                issue_gather(r + 1, obuf, osem)

            for j in range(_N_CTX):
                for i in range(_DIM // _LANES):
                    buf[1 + j, pl.ds(i * _LANES, _LANES)] = (
                        ctx_v[pl.ds(j * _DIM + i * _LANES, _LANES)])

            c = r * nw + wid
            pltpu.sync_copy(buf, out_hbm.at[c])

        issue_gather(0, row0, gs0)

        def body(rr, _):
            r0 = rr * 2
            round_sect(r0, row0, gs0, row1, gs1)

            @pl.when(r0 + 1 < nr)
            def _():
                round_sect(r0 + 1, row1, gs1, row0, gs0)

            return _

        lax.fori_loop(0, (rpw + 1) // 2, body, None)

    return k(idx_flat, table, ctx_flat)


def kernel(tokenized_prompts, token_embedding, ctx):
    n_cls = tokenized_prompts.shape[0]
    info = plsc.get_sparse_core_info()
    nw = info.num_cores * info.num_subcores
    rpw = -(-n_cls // nw)
    pad = nw * rpw - n_cls

    tokp = jnp.concatenate(
        [tokenized_prompts,
         jnp.zeros((pad, tokenized_prompts.shape[1]), jnp.int32)], axis=0)
    by_worker = tokp.reshape(rpw, nw, _SEQ).transpose(1, 0, 2)
    idx_flat = jnp.concatenate(
        [by_worker, jnp.zeros((nw, rpw, _IDXROW - _SEQ), jnp.int32)],
        axis=2).reshape(-1)

    return _sc_prompt_gather(idx_flat, token_embedding, ctx.reshape(-1),
                             n_cls)
